# baseline (device time: 41540 ns/iter reference)
import os

import jax
import jax.numpy as jnp
from jax import lax
from jax.experimental import pallas as pl
from jax.experimental.pallas import tpu as pltpu

_SKIP_RING = bool(os.environ.get("SKIP_RING"))

N_Y = 4
B, S, H, Dh, Dr = 2, 256, 16, 64, 32
D = 1024
DH2 = D // 2
BS = B * S


def kernel(x, Wdkv, Wuk, Wuv, Wq, Wqr, Wkr, Wo):
    d_c = Wdkv.shape[1]

    def body(x_ref, wdkv_ref, wuk_ref, wuv_ref, wq_ref, wqr_ref, wkr_ref,
             wo_ref, out_ref, cbuf, kwbufA, kwbufB, vwbuf,
             send_sems, recv_sems):
        my_x = lax.axis_index("x")
        my_y = lax.axis_index("y")
        my_z = lax.axis_index("z")

        if not _SKIP_RING:
            barrier_sem = pltpu.get_barrier_semaphore()
            for d in range(1, N_Y):
                pl.semaphore_signal(
                    barrier_sem, inc=1,
                    device_id=(my_x, lax.rem(my_y + d, N_Y), my_z),
                    device_id_type=pl.DeviceIdType.MESH,
                )
            pl.semaphore_wait(barrier_sem, N_Y - 1)

        xf = x_ref[...].reshape(BS, D).astype(jnp.bfloat16)
        c = jnp.dot(xf, wdkv_ref[...].astype(jnp.bfloat16),
                    preferred_element_type=jnp.float32).astype(jnp.bfloat16)
        cbuf[0] = c
        wuk_bf = wuk_ref[...].astype(jnp.bfloat16)
        kwbufA[0] = wuk_bf[:, :DH2]
        kwbufB[0] = wuk_bf[:, DH2:]
        vwbuf[0] = wuv_ref[...].astype(jnp.bfloat16)

        def start_peer_sends(t, buf):
            rdmas = []
            for d in range(1, N_Y):
                rdma = pltpu.make_async_remote_copy(
                    src_ref=buf.at[0],
                    dst_ref=buf.at[d],
                    send_sem=send_sems.at[d - 1, t],
                    recv_sem=recv_sems.at[d - 1, t],
                    device_id=(my_x, lax.rem(my_y + d, N_Y), my_z),
                    device_id_type=pl.DeviceIdType.MESH,
                )
                rdma.start()
                rdmas.append(rdma)
            return rdmas

        def wait_all(rdmas):
            for rdma in rdmas:
                rdma.wait()

        rdmas_A = []
        if not _SKIP_RING:
            rdmas_A = start_peer_sends(0, cbuf) + start_peer_sends(1, kwbufA)

        scale = (Dh + Dr) ** -0.5
        q2d = (jnp.dot(xf, wq_ref[...].astype(jnp.bfloat16),
                       preferred_element_type=jnp.float32) * scale
               ).astype(jnp.bfloat16)
        qr2d = (jnp.dot(xf, wqr_ref[...].astype(jnp.bfloat16),
                        preferred_element_type=jnp.float32) * scale
                ).astype(jnp.bfloat16)
        kr2d = jnp.dot(xf, wkr_ref[...].astype(jnp.bfloat16),
                       preferred_element_type=jnp.float32
                       ).astype(jnp.bfloat16)
        s2 = [[lax.dot_general(
                   qr2d[b * S:(b + 1) * S, h * Dr:(h + 1) * Dr],
                   kr2d[b * S:(b + 1) * S, :],
                   (((1,), (1,)), ((), ())),
                   preferred_element_type=jnp.float32)
               for h in range(H)] for b in range(B)]
        wo_bf = wo_ref[...].astype(jnp.bfloat16)

        wait_all(rdmas_A)
        rdmas_B = [] if _SKIP_RING else start_peer_sends(2, kwbufB)

        c_full = jnp.concatenate([cbuf[s] for s in range(N_Y)], axis=1)

        p_all = [[None] * H for _ in range(B)]
        zinv_all = [[None] * H for _ in range(B)]

        def scores_for(k2d_half, h0):
            for b in range(B):
                r0, r1 = b * S, (b + 1) * S
                for hh in range(H // 2):
                    h = h0 + hh
                    qh = q2d[r0:r1, h * Dh:(h + 1) * Dh]
                    kh = k2d_half[r0:r1, hh * Dh:(hh + 1) * Dh]
                    sc = lax.dot_general(qh, kh, (((1,), (1,)), ((), ())),
                                         preferred_element_type=jnp.float32
                                         ) + s2[b][h]
                    p = jnp.exp(sc)
                    z = jnp.sum(p, axis=-1, keepdims=True)
                    p_all[b][h] = p.astype(jnp.bfloat16)
                    zinv_all[b][h] = 1.0 / z

        wukA_full = jnp.concatenate([kwbufA[s] for s in range(N_Y)], axis=0)
        k2dA = jnp.dot(c_full, wukA_full,
                       preferred_element_type=jnp.float32
                       ).astype(jnp.bfloat16)
        scores_for(k2dA, 0)

        wait_all(rdmas_B)
        rdmas_C = [] if _SKIP_RING else start_peer_sends(3, vwbuf)

        wukB_full = jnp.concatenate([kwbufB[s] for s in range(N_Y)], axis=0)
        k2dB = jnp.dot(c_full, wukB_full,
                       preferred_element_type=jnp.float32
                       ).astype(jnp.bfloat16)
        scores_for(k2dB, H // 2)

        wait_all(rdmas_C)
        wuv_full = jnp.concatenate([vwbuf[s] for s in range(N_Y)], axis=0)
        v2d = jnp.dot(c_full, wuv_full,
                      preferred_element_type=jnp.float32
                      ).astype(jnp.bfloat16)

        for b in range(B):
            r0, r1 = b * S, (b + 1) * S
            o_heads = []
            for h in range(H):
                vh = v2d[r0:r1, h * Dh:(h + 1) * Dh]
                oh = lax.dot_general(p_all[b][h], vh,
                                     (((1,), (0,)), ((), ())),
                                     preferred_element_type=jnp.float32)
                oh = oh * zinv_all[b][h]
                o_heads.append(oh.astype(jnp.bfloat16))
            o_b = jnp.concatenate(o_heads, axis=-1)
            out_ref[b] = jnp.dot(o_b, wo_bf,
                                 preferred_element_type=jnp.float32)

    out_shape = jax.ShapeDtypeStruct((B, S, D), jnp.float32)
    return pl.pallas_call(
        body,
        out_shape=out_shape,
        in_specs=[pl.BlockSpec(memory_space=pltpu.VMEM)] * 8,
        out_specs=pl.BlockSpec(memory_space=pltpu.VMEM),
        scratch_shapes=[
            pltpu.VMEM((N_Y, BS, d_c), jnp.bfloat16),
            pltpu.VMEM((N_Y, d_c, DH2), jnp.bfloat16),
            pltpu.VMEM((N_Y, d_c, DH2), jnp.bfloat16),
            pltpu.VMEM((N_Y, d_c, D), jnp.bfloat16),
            pltpu.SemaphoreType.DMA((N_Y - 1, 4)),
            pltpu.SemaphoreType.DMA((N_Y - 1, 4)),
        ],
        compiler_params=(None if _SKIP_RING
                         else pltpu.CompilerParams(collective_id=0)),
    )(x, Wdkv, Wuk, Wuv, Wq, Wqr, Wkr, Wo)


# device time: 38745 ns/iter; 1.0721x vs baseline; 1.0721x over previous
import os

import jax
import jax.numpy as jnp
from jax import lax
from jax.experimental import pallas as pl
from jax.experimental.pallas import tpu as pltpu

_SKIP_RING = bool(os.environ.get("SKIP_RING"))

N_Y = 4
B, S, H, Dh, Dr = 2, 256, 16, 64, 32
D = 1024
BS = B * S


def kernel(x, Wdkv, Wuk, Wuv, Wq, Wqr, Wkr, Wo):
    d_c = Wdkv.shape[1]

    def body(x_ref, wdkv_ref, wuk_ref, wuv_ref, wq_ref, wqr_ref, wkr_ref,
             wo_ref, out_ref, cbuf, kwbuf, vwbuf, send_sems, recv_sems):
        my_x = lax.axis_index("x")
        my_y = lax.axis_index("y")
        my_z = lax.axis_index("z")

        if not _SKIP_RING:
            barrier_sem = pltpu.get_barrier_semaphore()
            for d in range(1, N_Y):
                pl.semaphore_signal(
                    barrier_sem, inc=1,
                    device_id=(my_x, lax.rem(my_y + d, N_Y), my_z),
                    device_id_type=pl.DeviceIdType.MESH,
                )
            pl.semaphore_wait(barrier_sem, N_Y - 1)

        xf = x_ref[...].reshape(BS, D).astype(jnp.bfloat16)
        c = jnp.dot(xf, wdkv_ref[...].astype(jnp.bfloat16),
                    preferred_element_type=jnp.float32).astype(jnp.bfloat16)
        cbuf[0] = c
        kwbuf[0] = wuk_ref[...].astype(jnp.bfloat16)
        vwbuf[0] = wuv_ref[...].astype(jnp.bfloat16)

        def start_peer_sends(t, buf):
            rdmas = []
            for d in range(1, N_Y):
                rdma = pltpu.make_async_remote_copy(
                    src_ref=buf.at[0],
                    dst_ref=buf.at[d],
                    send_sem=send_sems.at[d - 1, t],
                    recv_sem=recv_sems.at[d - 1, t],
                    device_id=(my_x, lax.rem(my_y + d, N_Y), my_z),
                    device_id_type=pl.DeviceIdType.MESH,
                )
                rdma.start()
                rdmas.append(rdma)
            return rdmas

        rdmas_ck = []
        if not _SKIP_RING:
            rdmas_ck = start_peer_sends(0, cbuf) + start_peer_sends(1, kwbuf)

        scale = (Dh + Dr) ** -0.5
        q2d = (jnp.dot(xf, wq_ref[...].astype(jnp.bfloat16),
                       preferred_element_type=jnp.float32) * scale
               ).astype(jnp.bfloat16)
        qr2d = (jnp.dot(xf, wqr_ref[...].astype(jnp.bfloat16),
                        preferred_element_type=jnp.float32) * scale
                ).astype(jnp.bfloat16)
        kr2d = jnp.dot(xf, wkr_ref[...].astype(jnp.bfloat16),
                       preferred_element_type=jnp.float32
                       ).astype(jnp.bfloat16)
        s2 = [[lax.dot_general(
                   qr2d[b * S:(b + 1) * S, h * Dr:(h + 1) * Dr],
                   kr2d[b * S:(b + 1) * S, :],
                   (((1,), (1,)), ((), ())),
                   preferred_element_type=jnp.float32)
               for h in range(H)] for b in range(B)]
        wo_bf = wo_ref[...].astype(jnp.bfloat16)

        for rdma in rdmas_ck:
            rdma.wait()
        rdmas_v = [] if _SKIP_RING else start_peer_sends(2, vwbuf)

        c_full = jnp.concatenate([cbuf[s] for s in range(N_Y)], axis=1)
        wuk_full = jnp.concatenate([kwbuf[s] for s in range(N_Y)], axis=0)
        k2d = jnp.dot(c_full, wuk_full,
                      preferred_element_type=jnp.float32
                      ).astype(jnp.bfloat16)

        p_all = [[None] * H for _ in range(B)]
        zinv_all = [[None] * H for _ in range(B)]
        for b in range(B):
            r0, r1 = b * S, (b + 1) * S
            for h in range(H):
                qh = q2d[r0:r1, h * Dh:(h + 1) * Dh]
                kh = k2d[r0:r1, h * Dh:(h + 1) * Dh]
                sc = lax.dot_general(qh, kh, (((1,), (1,)), ((), ())),
                                     preferred_element_type=jnp.float32
                                     ) + s2[b][h]
                p = jnp.exp(sc)
                z = jnp.sum(p, axis=-1, keepdims=True)
                p_all[b][h] = p.astype(jnp.bfloat16)
                zinv_all[b][h] = 1.0 / z

        for rdma in rdmas_v:
            rdma.wait()
        wuv_full = jnp.concatenate([vwbuf[s] for s in range(N_Y)], axis=0)
        v2d = jnp.dot(c_full, wuv_full,
                      preferred_element_type=jnp.float32
                      ).astype(jnp.bfloat16)

        for b in range(B):
            r0, r1 = b * S, (b + 1) * S
            o_heads = []
            for h in range(H):
                vh = v2d[r0:r1, h * Dh:(h + 1) * Dh]
                oh = lax.dot_general(p_all[b][h], vh,
                                     (((1,), (0,)), ((), ())),
                                     preferred_element_type=jnp.float32)
                oh = oh * zinv_all[b][h]
                o_heads.append(oh.astype(jnp.bfloat16))
            o_b = jnp.concatenate(o_heads, axis=-1)
            out_ref[b] = jnp.dot(o_b, wo_bf,
                                 preferred_element_type=jnp.float32)

    out_shape = jax.ShapeDtypeStruct((B, S, D), jnp.float32)
    return pl.pallas_call(
        body,
        out_shape=out_shape,
        in_specs=[pl.BlockSpec(memory_space=pltpu.VMEM)] * 8,
        out_specs=pl.BlockSpec(memory_space=pltpu.VMEM),
        scratch_shapes=[
            pltpu.VMEM((N_Y, BS, d_c), jnp.bfloat16),
            pltpu.VMEM((N_Y, d_c, D), jnp.bfloat16),
            pltpu.VMEM((N_Y, d_c, D), jnp.bfloat16),
            pltpu.SemaphoreType.DMA((N_Y - 1, 3)),
            pltpu.SemaphoreType.DMA((N_Y - 1, 3)),
        ],
        compiler_params=(None if _SKIP_RING
                         else pltpu.CompilerParams(collective_id=0)),
    )(x, Wdkv, Wuk, Wuv, Wq, Wqr, Wkr, Wo)


# device time: 36318 ns/iter; 1.1438x vs baseline; 1.0668x over previous
import os

import jax
import jax.numpy as jnp
from jax import lax
from jax.experimental import pallas as pl
from jax.experimental.pallas import tpu as pltpu

_SKIP_RING = bool(os.environ.get("SKIP_RING"))

N_Y = 4
B, S, H, Dh, Dr = 2, 256, 16, 64, 32
D = 1024
BS = B * S


def kernel(x, Wdkv, Wuk, Wuv, Wq, Wqr, Wkr, Wo):
    d_c = Wdkv.shape[1]

    def body(x_ref, wdkv_ref, wuk_ref, wuv_ref, wq_ref, wqr_ref, wkr_ref,
             wo_ref, out_ref, cbuf, kwbuf, vwbuf, send_sems, recv_sems):
        my_x = lax.axis_index("x")
        my_y = lax.axis_index("y")
        my_z = lax.axis_index("z")

        if not _SKIP_RING:
            barrier_sem = pltpu.get_barrier_semaphore()
            for d in range(1, N_Y):
                pl.semaphore_signal(
                    barrier_sem, inc=1,
                    device_id=(my_x, lax.rem(my_y + d, N_Y), my_z),
                    device_id_type=pl.DeviceIdType.MESH,
                )
            pl.semaphore_wait(barrier_sem, N_Y - 1)

        xf = x_ref[...].reshape(BS, D).astype(jnp.bfloat16)
        c = jnp.dot(xf, wdkv_ref[...].astype(jnp.bfloat16),
                    preferred_element_type=jnp.float32).astype(jnp.bfloat16)
        cbuf[0] = c
        kwbuf[0] = (wuk_ref[...] * 16.0).astype(jnp.float8_e4m3fn)
        vwbuf[0] = wuv_ref[...].astype(jnp.bfloat16)

        def start_peer_sends(t, buf):
            rdmas = []
            for d in range(1, N_Y):
                rdma = pltpu.make_async_remote_copy(
                    src_ref=buf.at[0],
                    dst_ref=buf.at[d],
                    send_sem=send_sems.at[d - 1, t],
                    recv_sem=recv_sems.at[d - 1, t],
                    device_id=(my_x, lax.rem(my_y + d, N_Y), my_z),
                    device_id_type=pl.DeviceIdType.MESH,
                )
                rdma.start()
                rdmas.append(rdma)
            return rdmas

        rdmas_ck = []
        if not _SKIP_RING:
            rdmas_ck = start_peer_sends(0, cbuf) + start_peer_sends(1, kwbuf)

        scale = (Dh + Dr) ** -0.5
        q2d = (jnp.dot(xf, wq_ref[...].astype(jnp.bfloat16),
                       preferred_element_type=jnp.float32) * (scale / 16.0)
               ).astype(jnp.bfloat16)
        qr2d = (jnp.dot(xf, wqr_ref[...].astype(jnp.bfloat16),
                        preferred_element_type=jnp.float32) * scale
                ).astype(jnp.bfloat16)
        kr2d = jnp.dot(xf, wkr_ref[...].astype(jnp.bfloat16),
                       preferred_element_type=jnp.float32
                       ).astype(jnp.bfloat16)
        s2 = [[lax.dot_general(
                   qr2d[b * S:(b + 1) * S, h * Dr:(h + 1) * Dr],
                   kr2d[b * S:(b + 1) * S, :],
                   (((1,), (1,)), ((), ())),
                   preferred_element_type=jnp.float32)
               for h in range(H)] for b in range(B)]
        wo_bf = wo_ref[...].astype(jnp.bfloat16)

        for rdma in rdmas_ck:
            rdma.wait()
        rdmas_v = [] if _SKIP_RING else start_peer_sends(2, vwbuf)

        c_full = jnp.concatenate([cbuf[s] for s in range(N_Y)], axis=1)
        wuk_full = jnp.concatenate(
            [kwbuf[s].astype(jnp.bfloat16) for s in range(N_Y)], axis=0)
        k2d = jnp.dot(c_full, wuk_full,
                      preferred_element_type=jnp.float32
                      ).astype(jnp.bfloat16)

        p_all = [[None] * H for _ in range(B)]
        zinv_all = [[None] * H for _ in range(B)]
        for b in range(B):
            r0, r1 = b * S, (b + 1) * S
            for h in range(H):
                qh = q2d[r0:r1, h * Dh:(h + 1) * Dh]
                kh = k2d[r0:r1, h * Dh:(h + 1) * Dh]
                sc = lax.dot_general(qh, kh, (((1,), (1,)), ((), ())),
                                     preferred_element_type=jnp.float32
                                     ) + s2[b][h]
                p = jnp.exp(sc)
                z = jnp.sum(p, axis=-1, keepdims=True)
                p_all[b][h] = p.astype(jnp.bfloat16)
                zinv_all[b][h] = 1.0 / z

        for rdma in rdmas_v:
            rdma.wait()
        wuv_full = jnp.concatenate([vwbuf[s] for s in range(N_Y)], axis=0)
        v2d = jnp.dot(c_full, wuv_full,
                      preferred_element_type=jnp.float32
                      ).astype(jnp.bfloat16)

        for b in range(B):
            r0, r1 = b * S, (b + 1) * S
            o_heads = []
            for h in range(H):
                vh = v2d[r0:r1, h * Dh:(h + 1) * Dh]
                oh = lax.dot_general(p_all[b][h], vh,
                                     (((1,), (0,)), ((), ())),
                                     preferred_element_type=jnp.float32)
                oh = oh * zinv_all[b][h]
                o_heads.append(oh.astype(jnp.bfloat16))
            o_b = jnp.concatenate(o_heads, axis=-1)
            out_ref[b] = jnp.dot(o_b, wo_bf,
                                 preferred_element_type=jnp.float32)

    out_shape = jax.ShapeDtypeStruct((B, S, D), jnp.float32)
    return pl.pallas_call(
        body,
        out_shape=out_shape,
        in_specs=[pl.BlockSpec(memory_space=pltpu.VMEM)] * 8,
        out_specs=pl.BlockSpec(memory_space=pltpu.VMEM),
        scratch_shapes=[
            pltpu.VMEM((N_Y, BS, d_c), jnp.bfloat16),
            pltpu.VMEM((N_Y, d_c, D), jnp.float8_e4m3fn),
            pltpu.VMEM((N_Y, d_c, D), jnp.bfloat16),
            pltpu.SemaphoreType.DMA((N_Y - 1, 3)),
            pltpu.SemaphoreType.DMA((N_Y - 1, 3)),
        ],
        compiler_params=(None if _SKIP_RING
                         else pltpu.CompilerParams(collective_id=0)),
    )(x, Wdkv, Wuk, Wuv, Wq, Wqr, Wkr, Wo)


# device time: 36310 ns/iter; 1.1440x vs baseline; 1.0002x over previous
import os

import jax
import jax.numpy as jnp
from jax import lax
from jax.experimental import pallas as pl
from jax.experimental.pallas import tpu as pltpu

_SKIP_RING = bool(os.environ.get("SKIP_RING"))

N_Y = 4
B, S, H, Dh, Dr = 2, 256, 16, 64, 32
D = 1024
BS = B * S


def kernel(x, Wdkv, Wuk, Wuv, Wq, Wqr, Wkr, Wo):
    d_c = Wdkv.shape[1]

    def body(x_ref, wdkv_ref, wuk_ref, wuv_ref, wq_ref, wqr_ref, wkr_ref,
             wo_ref, out_ref, cbuf, kwbuf, vwbuf, send_sems, recv_sems):
        my_x = lax.axis_index("x")
        my_y = lax.axis_index("y")
        my_z = lax.axis_index("z")

        if not _SKIP_RING:
            barrier_sem = pltpu.get_barrier_semaphore()
            for d in range(1, N_Y):
                pl.semaphore_signal(
                    barrier_sem, inc=1,
                    device_id=(my_x, lax.rem(my_y + d, N_Y), my_z),
                    device_id_type=pl.DeviceIdType.MESH,
                )
            pl.semaphore_wait(barrier_sem, N_Y - 1)

        xf = x_ref[...].reshape(BS, D).astype(jnp.bfloat16)
        c = jnp.dot(xf, wdkv_ref[...].astype(jnp.bfloat16),
                    preferred_element_type=jnp.float32).astype(jnp.bfloat16)
        cbuf[0] = c
        kwbuf[0] = (wuk_ref[...] * 16.0).astype(jnp.float8_e4m3fn)
        vwbuf[0] = wuv_ref[...].astype(jnp.bfloat16)

        def start_peer_sends(t, buf):
            rdmas = []
            for d in range(1, N_Y):
                rdma = pltpu.make_async_remote_copy(
                    src_ref=buf.at[0],
                    dst_ref=buf.at[d],
                    send_sem=send_sems.at[d - 1, t],
                    recv_sem=recv_sems.at[d - 1, t],
                    device_id=(my_x, lax.rem(my_y + d, N_Y), my_z),
                    device_id_type=pl.DeviceIdType.MESH,
                )
                rdma.start()
                rdmas.append(rdma)
            return rdmas

        rdmas_ck = []
        if not _SKIP_RING:
            rdmas_ck = start_peer_sends(0, cbuf) + start_peer_sends(1, kwbuf)

        scale = (Dh + Dr) ** -0.5 * 1.4426950408889634
        q2d = (jnp.dot(xf, wq_ref[...].astype(jnp.bfloat16),
                       preferred_element_type=jnp.float32) * (scale / 16.0)
               ).astype(jnp.bfloat16)
        qr2d = (jnp.dot(xf, wqr_ref[...].astype(jnp.bfloat16),
                        preferred_element_type=jnp.float32) * scale
                ).astype(jnp.bfloat16)
        kr2d = jnp.dot(xf, wkr_ref[...].astype(jnp.bfloat16),
                       preferred_element_type=jnp.float32
                       ).astype(jnp.bfloat16)
        s2 = [[lax.dot_general(
                   qr2d[b * S:(b + 1) * S, h * Dr:(h + 1) * Dr],
                   kr2d[b * S:(b + 1) * S, :],
                   (((1,), (1,)), ((), ())),
                   preferred_element_type=jnp.float32)
               for h in range(H)] for b in range(B)]
        wo_bf = wo_ref[...].astype(jnp.bfloat16)

        for rdma in rdmas_ck:
            rdma.wait()
        rdmas_v = [] if _SKIP_RING else start_peer_sends(2, vwbuf)

        c_full = jnp.concatenate([cbuf[s] for s in range(N_Y)], axis=1)
        wuk_full = jnp.concatenate(
            [kwbuf[s].astype(jnp.bfloat16) for s in range(N_Y)], axis=0)
        k2d = jnp.dot(c_full, wuk_full,
                      preferred_element_type=jnp.float32
                      ).astype(jnp.bfloat16)

        p_all = [[None] * H for _ in range(B)]
        zinv_all = [[None] * H for _ in range(B)]
        for b in range(B):
            r0, r1 = b * S, (b + 1) * S
            for h in range(H):
                qh = q2d[r0:r1, h * Dh:(h + 1) * Dh]
                kh = k2d[r0:r1, h * Dh:(h + 1) * Dh]
                sc = lax.dot_general(qh, kh, (((1,), (1,)), ((), ())),
                                     preferred_element_type=jnp.float32
                                     ) + s2[b][h]
                p = jnp.exp2(sc)
                z = jnp.sum(p, axis=-1, keepdims=True)
                p_all[b][h] = p.astype(jnp.bfloat16)
                zinv_all[b][h] = 1.0 / z

        for rdma in rdmas_v:
            rdma.wait()
        wuv_full = jnp.concatenate([vwbuf[s] for s in range(N_Y)], axis=0)
        v2d = jnp.dot(c_full, wuv_full,
                      preferred_element_type=jnp.float32
                      ).astype(jnp.bfloat16)

        for b in range(B):
            r0, r1 = b * S, (b + 1) * S
            o_heads = []
            for h in range(H):
                vh = v2d[r0:r1, h * Dh:(h + 1) * Dh]
                oh = lax.dot_general(p_all[b][h], vh,
                                     (((1,), (0,)), ((), ())),
                                     preferred_element_type=jnp.float32)
                oh = oh * zinv_all[b][h]
                o_heads.append(oh.astype(jnp.bfloat16))
            o_b = jnp.concatenate(o_heads, axis=-1)
            out_ref[b] = jnp.dot(o_b, wo_bf,
                                 preferred_element_type=jnp.float32)

    out_shape = jax.ShapeDtypeStruct((B, S, D), jnp.float32)
    return pl.pallas_call(
        body,
        out_shape=out_shape,
        in_specs=[pl.BlockSpec(memory_space=pltpu.VMEM)] * 8,
        out_specs=pl.BlockSpec(memory_space=pltpu.VMEM),
        scratch_shapes=[
            pltpu.VMEM((N_Y, BS, d_c), jnp.bfloat16),
            pltpu.VMEM((N_Y, d_c, D), jnp.float8_e4m3fn),
            pltpu.VMEM((N_Y, d_c, D), jnp.bfloat16),
            pltpu.SemaphoreType.DMA((N_Y - 1, 3)),
            pltpu.SemaphoreType.DMA((N_Y - 1, 3)),
        ],
        compiler_params=(None if _SKIP_RING
                         else pltpu.CompilerParams(collective_id=0)),
    )(x, Wdkv, Wuk, Wuv, Wq, Wqr, Wkr, Wo)


# device time: 35902 ns/iter; 1.1570x vs baseline; 1.0114x over previous
import os

import jax
import jax.numpy as jnp
from jax import lax
from jax.experimental import pallas as pl
from jax.experimental.pallas import tpu as pltpu

_SKIP_RING = bool(os.environ.get("SKIP_RING"))

N_Y = 4
B, S, H, Dh, Dr = 2, 256, 16, 64, 32
D = 1024
BS = B * S


def kernel(x, Wdkv, Wuk, Wuv, Wq, Wqr, Wkr, Wo):
    d_c = Wdkv.shape[1]

    def body(x_ref, wdkv_ref, wuk_ref, wuv_ref, wq_ref, wqr_ref, wkr_ref,
             wo_ref, out_ref, cbuf, kwbuf, vwbuf, send_sems, recv_sems):
        my_x = lax.axis_index("x")
        my_y = lax.axis_index("y")
        my_z = lax.axis_index("z")

        if not _SKIP_RING:
            barrier_sem = pltpu.get_barrier_semaphore()
            for d in range(1, N_Y):
                pl.semaphore_signal(
                    barrier_sem, inc=1,
                    device_id=(my_x, lax.rem(my_y + d, N_Y), my_z),
                    device_id_type=pl.DeviceIdType.MESH,
                )
            pl.semaphore_wait(barrier_sem, N_Y - 1)

        xf = x_ref[...].reshape(BS, D).astype(jnp.bfloat16)
        c = jnp.dot(xf, wdkv_ref[...].astype(jnp.bfloat16),
                    preferred_element_type=jnp.float32).astype(jnp.bfloat16)
        cbuf[0] = c
        kwbuf[0] = (wuk_ref[...] * 16.0).astype(jnp.float8_e4m3fn)
        vwbuf[0] = wuv_ref[...].astype(jnp.bfloat16)

        def start_peer_sends(t, buf):
            rdmas = []
            for d in range(1, N_Y):
                rdma = pltpu.make_async_remote_copy(
                    src_ref=buf.at[0],
                    dst_ref=buf.at[d],
                    send_sem=send_sems.at[d - 1, t],
                    recv_sem=recv_sems.at[d - 1, t],
                    device_id=(my_x, lax.rem(my_y + d, N_Y), my_z),
                    device_id_type=pl.DeviceIdType.MESH,
                )
                rdma.start()
                rdmas.append(rdma)
            return rdmas

        rdmas_ck = []
        if not _SKIP_RING:
            rdmas_ck = start_peer_sends(0, cbuf) + start_peer_sends(1, kwbuf)

        scale = (Dh + Dr) ** -0.5 * 1.4426950408889634
        q2d = (jnp.dot(xf, wq_ref[...].astype(jnp.bfloat16),
                       preferred_element_type=jnp.float32) * (scale / 16.0)
               ).astype(jnp.bfloat16)
        qr2d = (jnp.dot(xf, wqr_ref[...].astype(jnp.bfloat16),
                        preferred_element_type=jnp.float32) * scale
                ).astype(jnp.bfloat16)
        kr2d = jnp.dot(xf, wkr_ref[...].astype(jnp.bfloat16),
                       preferred_element_type=jnp.float32
                       ).astype(jnp.bfloat16)
        s2 = [[lax.dot_general(
                   qr2d[b * S:(b + 1) * S, h * Dr:(h + 1) * Dr],
                   kr2d[b * S:(b + 1) * S, :],
                   (((1,), (1,)), ((), ())),
                   preferred_element_type=jnp.float32)
               for h in range(H)] for b in range(B)]
        wo_bf = wo_ref[...].astype(jnp.bfloat16)

        for rdma in rdmas_ck:
            rdma.wait()
        rdmas_v = [] if _SKIP_RING else start_peer_sends(2, vwbuf)

        c_full = jnp.concatenate([cbuf[s] for s in range(N_Y)], axis=1)
        wuk_full = jnp.concatenate(
            [kwbuf[s].astype(jnp.bfloat16) for s in range(N_Y)], axis=0)
        k2d = jnp.dot(c_full, wuk_full,
                      preferred_element_type=jnp.float32
                      ).astype(jnp.bfloat16)

        p_all = [[None] * H for _ in range(B)]
        zinv_all = [[None] * H for _ in range(B)]
        for b in range(B):
            r0, r1 = b * S, (b + 1) * S
            for h in range(H):
                qh = q2d[r0:r1, h * Dh:(h + 1) * Dh]
                kh = k2d[r0:r1, h * Dh:(h + 1) * Dh]
                sc = lax.dot_general(qh, kh, (((1,), (1,)), ((), ())),
                                     preferred_element_type=jnp.float32
                                     ) + s2[b][h]
                p = jnp.exp2(sc)
                z = jnp.sum(p, axis=-1, keepdims=True)
                p_all[b][h] = p.astype(jnp.bfloat16)
                zinv_all[b][h] = 1.0 / z

        for rdma in rdmas_v:
            rdma.wait()
        wuv_full = jnp.concatenate([vwbuf[s] for s in range(N_Y)], axis=0)
        v2d = jnp.dot(c_full, wuv_full,
                      preferred_element_type=jnp.float32
                      ).astype(jnp.bfloat16)

        for b in range(B):
            r0, r1 = b * S, (b + 1) * S
            o_heads = []
            for h in range(H):
                vh = v2d[r0:r1, h * Dh:(h + 1) * Dh]
                oh = lax.dot_general(p_all[b][h], vh,
                                     (((1,), (0,)), ((), ())),
                                     preferred_element_type=jnp.float32)
                oh = oh * zinv_all[b][h]
                o_heads.append(oh.astype(jnp.bfloat16))
            o_b = jnp.concatenate(o_heads, axis=-1)
            out_ref[b] = jnp.dot(o_b, wo_bf,
                                 preferred_element_type=jnp.float32
                                 ).astype(jnp.bfloat16)

    out_shape = jax.ShapeDtypeStruct((B, S, D), jnp.bfloat16)
    return pl.pallas_call(
        body,
        out_shape=out_shape,
        in_specs=[pl.BlockSpec(memory_space=pltpu.VMEM)] * 8,
        out_specs=pl.BlockSpec(memory_space=pltpu.VMEM),
        scratch_shapes=[
            pltpu.VMEM((N_Y, BS, d_c), jnp.bfloat16),
            pltpu.VMEM((N_Y, d_c, D), jnp.float8_e4m3fn),
            pltpu.VMEM((N_Y, d_c, D), jnp.bfloat16),
            pltpu.SemaphoreType.DMA((N_Y - 1, 3)),
            pltpu.SemaphoreType.DMA((N_Y - 1, 3)),
        ],
        compiler_params=(None if _SKIP_RING
                         else pltpu.CompilerParams(collective_id=0)),
    )(x, Wdkv, Wuk, Wuv, Wq, Wqr, Wkr, Wo)
